# Initial kernel scaffold; baseline (speedup 1.0000x reference)
import jax
import jax.numpy as jnp
from jax.experimental import pallas as pl


def kernel(x, edge_index, W, b):
    raise NotImplementedError("write your pallas kernel here")


# speedup vs baseline: 21.8939x; 21.8939x over previous
import functools

import jax
import jax.numpy as jnp
from jax import lax
from jax.experimental import pallas as pl
from jax.experimental.pallas import tpu as pltpu
from jax.experimental.pallas import tpu_sc as plsc

_NC = 2
_NS = 16
_CH = 128
_LANES = 16


def _deg_body(dst_hbm, hist_out, dstv, onesv, zerosv, hist_sh):
    c = lax.axis_index("c")
    s = lax.axis_index("s")
    wid = c * _NS + s
    np_ = hist_sh.shape[0]
    per_tile = np_ // _NS

    def fill_zeros(i, carry):
        zerosv[pl.ds(i * _LANES, _LANES)] = jnp.zeros((_LANES,), jnp.float32)
        return carry

    lax.fori_loop(0, per_tile // _LANES, fill_zeros, 0)

    def fill_ones(i, carry):
        onesv[pl.ds(i * _LANES, _LANES)] = jnp.ones((_LANES,), jnp.float32)
        return carry

    lax.fori_loop(0, _CH // _LANES, fill_ones, 0)

    pltpu.sync_copy(zerosv, hist_sh.at[pl.ds(s * per_tile, per_tile)])
    plsc.subcore_barrier()

    rows_per_w = dst_hbm.shape[0] // (_NC * _NS)
    base = wid * rows_per_w

    def body(i, carry):
        pltpu.sync_copy(dst_hbm.at[base + i], dstv)
        pltpu.sync_copy(onesv, hist_sh.at[dstv], add=True)
        return carry

    lax.fori_loop(0, rows_per_w, body, 0)
    plsc.subcore_barrier()
    pltpu.sync_copy(hist_sh.at[pl.ds(s * per_tile, per_tile)],
                    hist_out.at[c].at[pl.ds(s * per_tile, per_tile)])


def _edge_body(src_hbm, dst_hbm, y_hbm, acc_out, srcv, dstv, rows, acc_sh):
    c = lax.axis_index("c")
    s = lax.axis_index("s")
    wid = c * _NS + s
    np_ = acc_sh.shape[0]
    per_tile = np_ // _NS

    pltpu.sync_copy(y_hbm.at[pl.ds(s * per_tile, per_tile)],
                    acc_sh.at[pl.ds(s * per_tile, per_tile)])
    plsc.subcore_barrier()

    rows_per_w = src_hbm.shape[0] // (_NC * _NS)
    base = wid * rows_per_w

    def body(i, carry):
        pltpu.sync_copy(src_hbm.at[base + i], srcv)
        pltpu.sync_copy(dst_hbm.at[base + i], dstv)
        pltpu.sync_copy(y_hbm.at[srcv], rows)
        pltpu.sync_copy(rows, acc_sh.at[dstv], add=True)
        return carry

    lax.fori_loop(0, rows_per_w, body, 0)
    plsc.subcore_barrier()
    pltpu.sync_copy(acc_sh.at[pl.ds(s * per_tile, per_tile)],
                    acc_out.at[c].at[pl.ds(s * per_tile, per_tile)])


def _tc1_body(x_ref, w_ref, da_ref, db_ref, y_ref, dis_ref):
    deg = da_ref[...] + db_ref[...] + 1.0
    dis = lax.rsqrt(deg)
    xw = jnp.dot(x_ref[...], w_ref[...], preferred_element_type=jnp.float32)
    y_ref[...] = xw * dis
    dis_ref[...] = dis


def _tc2_body(acc_ref, y_ref, dis_ref, b_ref, o_ref):
    total = acc_ref[0] + acc_ref[1] - y_ref[...]
    o_ref[...] = jnp.maximum(dis_ref[...] * total + b_ref[...], 0.0)


def kernel(x, edge_index, W, b):
    N, D = x.shape
    E = edge_index.shape[1]
    NW = _NC * _NS
    per_w = -(-E // (NW * _CH))
    Ep = NW * per_w * _CH
    pad = Ep - E
    NP = (-(-N // _CH)) * _CH + _CH

    ar = jnp.arange(pad, dtype=jnp.int32)
    src2d = jnp.concatenate([edge_index[0], ar % N]).reshape(NW * per_w, _CH)
    dst2d = jnp.concatenate([edge_index[1], N + ar % (NP - N)]).reshape(
        NW * per_w, _CH)

    mesh = plsc.VectorSubcoreMesh(core_axis_name="c", subcore_axis_name="s",
                                  num_cores=_NC, num_subcores=_NS)

    hist = pl.kernel(
        _deg_body,
        out_type=jax.ShapeDtypeStruct((_NC, NP), jnp.float32),
        mesh=mesh,
        scratch_types=[
            pltpu.VMEM((_CH,), jnp.int32),
            pltpu.VMEM((_CH,), jnp.float32),
            pltpu.VMEM((NP // _NS,), jnp.float32),
            pltpu.VMEM_SHARED((NP,), jnp.float32),
        ],
    )(dst2d)

    da = hist[0].reshape(NP, 1)
    db = hist[1].reshape(NP, 1)

    BL = 1000
    grid = (N // BL,)
    y, dis = pl.pallas_call(
        _tc1_body,
        grid=grid,
        in_specs=[
            pl.BlockSpec((BL, D), lambda i: (i, 0)),
            pl.BlockSpec((D, D), lambda i: (0, 0)),
            pl.BlockSpec((BL, 1), lambda i: (i, 0)),
            pl.BlockSpec((BL, 1), lambda i: (i, 0)),
        ],
        out_specs=[
            pl.BlockSpec((BL, D), lambda i: (i, 0)),
            pl.BlockSpec((BL, 1), lambda i: (i, 0)),
        ],
        out_shape=[
            jax.ShapeDtypeStruct((NP, D), jnp.float32),
            jax.ShapeDtypeStruct((NP, 1), jnp.float32),
        ],
    )(x, W, da, db)

    acc = pl.kernel(
        _edge_body,
        out_type=jax.ShapeDtypeStruct((_NC, NP, D), jnp.float32),
        mesh=mesh,
        scratch_types=[
            pltpu.VMEM((_CH,), jnp.int32),
            pltpu.VMEM((_CH,), jnp.int32),
            pltpu.VMEM((_CH, D), jnp.float32),
            pltpu.VMEM_SHARED((NP, D), jnp.float32),
        ],
    )(src2d, dst2d, y)

    out = pl.pallas_call(
        _tc2_body,
        grid=grid,
        in_specs=[
            pl.BlockSpec((_NC, BL, D), lambda i: (0, i, 0)),
            pl.BlockSpec((BL, D), lambda i: (i, 0)),
            pl.BlockSpec((BL, 1), lambda i: (i, 0)),
            pl.BlockSpec((1, D), lambda i: (0, 0)),
        ],
        out_specs=pl.BlockSpec((BL, D), lambda i: (i, 0)),
        out_shape=jax.ShapeDtypeStruct((N, D), jnp.float32),
    )(acc, y, dis, b.reshape(1, D))

    return out


# speedup vs baseline: 42.4421x; 1.9385x over previous
import jax
import jax.numpy as jnp
from jax import lax
from jax.experimental import pallas as pl
from jax.experimental.pallas import tpu as pltpu
from jax.experimental.pallas import tpu_sc as plsc

_NC = 2
_NS = 16
_CH = 128
_LANES = 16
_NB = 4


def _deg_body(dst_hbm, hist_out, dstb, onesv, zerosv, hist_sh,
              sem0, sem1, sem2, sem3):
    c = lax.axis_index("c")
    s = lax.axis_index("s")
    np_ = hist_sh.shape[0]
    per_tile = np_ // _NS
    per_w = dstb.shape[0]
    sems = (sem0, sem1, sem2, sem3)

    def fill_zeros(i, carry):
        zerosv[pl.ds(i * _LANES, _LANES)] = jnp.zeros((_LANES,), jnp.float32)
        return carry

    lax.fori_loop(0, per_tile // _LANES, fill_zeros, 0)

    def fill_ones(i, carry):
        onesv[pl.ds(i * _LANES, _LANES)] = jnp.ones((_LANES,), jnp.float32)
        return carry

    lax.fori_loop(0, _CH // _LANES, fill_ones, 0)

    pltpu.sync_copy(zerosv, hist_sh.at[pl.ds(s * per_tile, per_tile)])
    pltpu.sync_copy(dst_hbm.at[s].at[pl.ds(c * per_w, per_w)], dstb)
    plsc.subcore_barrier()

    for k in range(_NB):
        pltpu.async_copy(onesv, hist_sh.at[dstb.at[k]], sems[k], add=True)

    def body(j, carry):
        for k in range(_NB):
            m = j * _NB + k
            pltpu.make_async_copy(onesv, hist_sh.at[dstb.at[m]],
                                  sems[k]).wait()
            pltpu.async_copy(onesv, hist_sh.at[dstb.at[m + _NB]],
                             sems[k], add=True)
        return carry

    lax.fori_loop(0, (per_w - _NB) // _NB, body, 0)
    for k in range(_NB):
        m = per_w - _NB + k
        pltpu.make_async_copy(onesv, hist_sh.at[dstb.at[m]],
                              sems[m % _NB]).wait()

    plsc.subcore_barrier()
    pltpu.sync_copy(hist_sh.at[pl.ds(s * per_tile, per_tile)],
                    hist_out.at[c].at[pl.ds(s * per_tile, per_tile)])


def _edge_body(src_hbm, dst_hbm, y_hbm, acc_out, srcb, dstb,
               rows0, rows1, rows2, rows3, acc_sh,
               gs0, gs1, gs2, gs3, ssem):
    c = lax.axis_index("c")
    s = lax.axis_index("s")
    np_ = acc_sh.shape[0]
    per_tile = np_ // _NS
    per_t = src_hbm.shape[2]
    rows = (rows0, rows1, rows2, rows3)
    gsems = (gs0, gs1, gs2, gs3)

    pltpu.sync_copy(y_hbm.at[pl.ds(c * np_ + s * per_tile, per_tile)],
                    acc_sh.at[pl.ds(s * per_tile, per_tile)])
    pltpu.sync_copy(src_hbm.at[c].at[s], srcb)
    pltpu.sync_copy(dst_hbm.at[s], dstb)
    plsc.subcore_barrier()

    for k in range(_NB - 1):
        pltpu.async_copy(y_hbm.at[srcb.at[k]], rows[k], gsems[k])

    def body(j, carry):
        for k in range(_NB):
            m = j * _NB + k
            kp = (k + _NB - 1) % _NB

            def wait_prev_scatter():
                pltpu.make_async_copy(rows[0], acc_sh.at[dstb.at[m - 1]],
                                      ssem).wait()

            if k == 0:
                pl.when(j >= 1)(wait_prev_scatter)
            else:
                wait_prev_scatter()

            pltpu.async_copy(y_hbm.at[srcb.at[m + _NB - 1]], rows[kp],
                             gsems[kp])
            pltpu.make_async_copy(y_hbm.at[srcb.at[m]], rows[k],
                                  gsems[k]).wait()
            pltpu.async_copy(rows[k], acc_sh.at[dstb.at[m]], ssem, add=True)
        return carry

    lax.fori_loop(0, (per_t - _NB) // _NB, body, 0)

    for m in range(per_t - _NB, per_t):
        km = m % _NB
        pltpu.make_async_copy(rows[0], acc_sh.at[dstb.at[m - 1]], ssem).wait()
        if m + _NB - 1 < per_t:
            kp = (m + _NB - 1) % _NB
            pltpu.async_copy(y_hbm.at[srcb.at[m + _NB - 1]], rows[kp],
                             gsems[kp])
        pltpu.make_async_copy(y_hbm.at[srcb.at[m]], rows[km],
                              gsems[km]).wait()
        pltpu.async_copy(rows[km], acc_sh.at[dstb.at[m]], ssem, add=True)
    pltpu.make_async_copy(rows[0], acc_sh.at[dstb.at[per_t - 1]], ssem).wait()

    plsc.subcore_barrier()
    pltpu.sync_copy(acc_sh.at[pl.ds(s * per_tile, per_tile)],
                    acc_out.at[c].at[pl.ds(s * per_tile, per_tile)])


def _tc1_body(x_ref, w_ref, da_ref, db_ref, y_ref, dis_ref):
    hd = y_ref.shape[2]
    deg = da_ref[...] + db_ref[...] + 1.0
    dis = lax.rsqrt(deg)
    xw = jnp.dot(x_ref[...], w_ref[...], preferred_element_type=jnp.float32)
    y_ref[0] = xw[:, :hd] * dis
    y_ref[1] = xw[:, hd:] * dis
    dis_ref[...] = dis


def _tc2_body(acc_ref, dis_ref, b_ref, o_ref):
    hd = acc_ref.shape[2]
    dis = dis_ref[...]
    o_ref[:, :hd] = jnp.maximum(dis * acc_ref[0] + b_ref[:, :hd], 0.0)
    o_ref[:, hd:] = jnp.maximum(dis * acc_ref[1] + b_ref[:, hd:], 0.0)


def kernel(x, edge_index, W, b):
    N, D = x.shape
    HD = D // _NC
    E = edge_index.shape[1]
    per_t = -(-E // (_NS * _CH))
    per_t = -(-per_t // (2 * _NB)) * (2 * _NB)
    Ep = _NS * per_t * _CH
    pad = Ep - E
    NP = (-(-N // _CH)) * _CH + _CH

    ar = jnp.arange(pad, dtype=jnp.int32)
    srcp = jnp.concatenate([edge_index[0], ar % N])
    dstp = jnp.concatenate([edge_index[1], N + ar % (NP - N)])
    src4d = jnp.stack([srcp, srcp + NP]).reshape(_NC, _NS, per_t, _CH)
    dst3d = dstp.reshape(_NS, per_t, _CH)

    mesh = plsc.VectorSubcoreMesh(core_axis_name="c", subcore_axis_name="s",
                                  num_cores=_NC, num_subcores=_NS)

    hist = pl.kernel(
        _deg_body,
        out_type=jax.ShapeDtypeStruct((_NC, NP), jnp.float32),
        mesh=mesh,
        scratch_types=[
            pltpu.VMEM((per_t // _NC, _CH), jnp.int32),
            pltpu.VMEM((_CH,), jnp.float32),
            pltpu.VMEM((NP // _NS,), jnp.float32),
            pltpu.VMEM_SHARED((NP,), jnp.float32),
        ] + [pltpu.SemaphoreType.DMA] * _NB,
    )(dst3d)

    da = hist[0].reshape(NP, 1)
    db = hist[1].reshape(NP, 1)

    BL = 1000
    grid = (N // BL,)
    y3, dis = pl.pallas_call(
        _tc1_body,
        grid=grid,
        in_specs=[
            pl.BlockSpec((BL, D), lambda i: (i, 0)),
            pl.BlockSpec((D, D), lambda i: (0, 0)),
            pl.BlockSpec((BL, 1), lambda i: (i, 0)),
            pl.BlockSpec((BL, 1), lambda i: (i, 0)),
        ],
        out_specs=[
            pl.BlockSpec((_NC, BL, HD), lambda i: (0, i, 0)),
            pl.BlockSpec((BL, 1), lambda i: (i, 0)),
        ],
        out_shape=[
            jax.ShapeDtypeStruct((_NC, NP, HD), jnp.float32),
            jax.ShapeDtypeStruct((NP, 1), jnp.float32),
        ],
    )(x, W, da, db)

    y2 = y3.reshape(_NC * NP, HD)

    acc = pl.kernel(
        _edge_body,
        out_type=jax.ShapeDtypeStruct((_NC, NP, HD), jnp.float32),
        mesh=mesh,
        scratch_types=[
            pltpu.VMEM((per_t, _CH), jnp.int32),
            pltpu.VMEM((per_t, _CH), jnp.int32),
            pltpu.VMEM((_CH, HD), jnp.float32),
            pltpu.VMEM((_CH, HD), jnp.float32),
            pltpu.VMEM((_CH, HD), jnp.float32),
            pltpu.VMEM((_CH, HD), jnp.float32),
            pltpu.VMEM_SHARED((NP, HD), jnp.float32),
        ] + [pltpu.SemaphoreType.DMA] * (_NB + 1),
        compiler_params=pltpu.CompilerParams(use_tc_tiling_on_sc=False),
    )(src4d, dst3d, y2)

    out = pl.pallas_call(
        _tc2_body,
        grid=grid,
        in_specs=[
            pl.BlockSpec((_NC, BL, HD), lambda i: (0, i, 0)),
            pl.BlockSpec((BL, 1), lambda i: (i, 0)),
            pl.BlockSpec((1, D), lambda i: (0, 0)),
        ],
        out_specs=pl.BlockSpec((BL, D), lambda i: (i, 0)),
        out_shape=jax.ShapeDtypeStruct((N, D), jnp.float32),
    )(acc, dis, b.reshape(1, D))

    return out


# speedup vs baseline: 43.0212x; 1.0136x over previous
import jax
import jax.numpy as jnp
from jax import lax
from jax.experimental import pallas as pl
from jax.experimental.pallas import tpu as pltpu
from jax.experimental.pallas import tpu_sc as plsc

_NC = 2
_NS = 16
_CH = 128
_LANES = 16
_NB = 4


def _deg_body(dst_hbm, hist_out, dstb, onesv, zerosv, hist_sh,
              sem0, sem1, sem2, sem3):
    c = lax.axis_index("c")
    s = lax.axis_index("s")
    np_ = hist_sh.shape[0]
    per_tile = np_ // _NS
    per_w = dstb.shape[0]
    sems = (sem0, sem1, sem2, sem3)

    def fill_zeros(i, carry):
        zerosv[pl.ds(i * _LANES, _LANES)] = jnp.zeros((_LANES,), jnp.float32)
        return carry

    lax.fori_loop(0, per_tile // _LANES, fill_zeros, 0)

    def fill_ones(i, carry):
        onesv[pl.ds(i * _LANES, _LANES)] = jnp.ones((_LANES,), jnp.float32)
        return carry

    lax.fori_loop(0, _CH // _LANES, fill_ones, 0)

    pltpu.sync_copy(zerosv, hist_sh.at[pl.ds(s * per_tile, per_tile)])
    pltpu.sync_copy(dst_hbm.at[s].at[pl.ds(c * per_w, per_w)], dstb)
    plsc.subcore_barrier()

    for k in range(_NB):
        pltpu.async_copy(onesv, hist_sh.at[dstb.at[k]], sems[k], add=True)

    def body(j, carry):
        for k in range(_NB):
            m = j * _NB + k
            pltpu.make_async_copy(onesv, hist_sh.at[dstb.at[m]],
                                  sems[k]).wait()
            pltpu.async_copy(onesv, hist_sh.at[dstb.at[m + _NB]],
                             sems[k], add=True)
        return carry

    lax.fori_loop(0, (per_w - _NB) // _NB, body, 0)
    for k in range(_NB):
        m = per_w - _NB + k
        pltpu.make_async_copy(onesv, hist_sh.at[dstb.at[m]],
                              sems[m % _NB]).wait()

    plsc.subcore_barrier()
    pltpu.sync_copy(hist_sh.at[pl.ds(s * per_tile, per_tile)],
                    hist_out.at[c].at[pl.ds(s * per_tile, per_tile)])


def _edge_body(src_hbm, dst_hbm, y_hbm, acc_out, srcb, dstb,
               rows0, rows1, rows2, rows3, acc_sh,
               gs0, gs1, gs2, gs3, ssem):
    c = lax.axis_index("c")
    s = lax.axis_index("s")
    np_ = acc_sh.shape[0]
    per_tile = np_ // _NS
    per_t = src_hbm.shape[2]
    rows = (rows0, rows1, rows2, rows3)
    gsems = (gs0, gs1, gs2, gs3)
    hd = rows0.shape[1]

    def fill_zero_rows(i, carry):
        for j in range(hd // _LANES):
            rows0[i, pl.ds(j * _LANES, _LANES)] = jnp.zeros((_LANES,),
                                                            jnp.float32)
        return carry

    lax.fori_loop(0, _CH, fill_zero_rows, 0)
    for k in range(per_tile // _CH):
        pltpu.sync_copy(rows0,
                        acc_sh.at[pl.ds(s * per_tile + k * _CH, _CH)])
    pltpu.sync_copy(src_hbm.at[c].at[s], srcb)
    pltpu.sync_copy(dst_hbm.at[s], dstb)
    plsc.subcore_barrier()

    for k in range(_NB - 1):
        pltpu.async_copy(y_hbm.at[srcb.at[k]], rows[k], gsems[k])

    def body(j, carry):
        for k in range(_NB):
            m = j * _NB + k
            kp = (k + _NB - 1) % _NB

            def wait_prev_scatter():
                pltpu.make_async_copy(rows[0], acc_sh.at[dstb.at[m - 1]],
                                      ssem).wait()

            if k == 0:
                pl.when(j >= 1)(wait_prev_scatter)
            else:
                wait_prev_scatter()

            pltpu.async_copy(y_hbm.at[srcb.at[m + _NB - 1]], rows[kp],
                             gsems[kp])
            pltpu.make_async_copy(y_hbm.at[srcb.at[m]], rows[k],
                                  gsems[k]).wait()
            pltpu.async_copy(rows[k], acc_sh.at[dstb.at[m]], ssem, add=True)
        return carry

    lax.fori_loop(0, (per_t - _NB) // _NB, body, 0)

    for m in range(per_t - _NB, per_t):
        km = m % _NB
        pltpu.make_async_copy(rows[0], acc_sh.at[dstb.at[m - 1]], ssem).wait()
        if m + _NB - 1 < per_t:
            kp = (m + _NB - 1) % _NB
            pltpu.async_copy(y_hbm.at[srcb.at[m + _NB - 1]], rows[kp],
                             gsems[kp])
        pltpu.make_async_copy(y_hbm.at[srcb.at[m]], rows[km],
                              gsems[km]).wait()
        pltpu.async_copy(rows[km], acc_sh.at[dstb.at[m]], ssem, add=True)
    pltpu.make_async_copy(rows[0], acc_sh.at[dstb.at[per_t - 1]], ssem).wait()

    plsc.subcore_barrier()
    pltpu.sync_copy(acc_sh.at[pl.ds(s * per_tile, per_tile)],
                    acc_out.at[c].at[pl.ds(s * per_tile, per_tile)])


def _tc1_body(x_ref, w_ref, da_ref, db_ref, y_ref, dis_ref):
    deg = da_ref[...] + db_ref[...]
    dis = lax.rsqrt(deg)
    xw = jnp.dot(x_ref[...], w_ref[...], preferred_element_type=jnp.float32)
    y_ref[...] = xw * dis
    dis_ref[...] = dis


def _tc2_body(acc_ref, dis_ref, b_ref, o_ref):
    hd = acc_ref.shape[2]
    dis = dis_ref[...]
    o_ref[:, :hd] = jnp.maximum(dis * acc_ref[0] + b_ref[:, :hd], 0.0)
    o_ref[:, hd:] = jnp.maximum(dis * acc_ref[1] + b_ref[:, hd:], 0.0)


def kernel(x, edge_index, W, b):
    N, D = x.shape
    HD = D // _NC
    E = edge_index.shape[1]
    E2 = E + N
    per_t = -(-E2 // (_NS * _CH))
    per_t = -(-per_t // (2 * _NB)) * (2 * _NB)
    Ep = _NS * per_t * _CH
    pad = Ep - E2
    NP = (-(-N // _CH)) * _CH + _CH

    loop = jnp.arange(N, dtype=jnp.int32)
    ar = jnp.arange(pad, dtype=jnp.int32)
    srcp = jnp.concatenate([edge_index[0], loop, ar % N])
    dstp = jnp.concatenate([edge_index[1], loop, N + ar % (NP - N)])
    src4d = jnp.stack([2 * srcp, 2 * srcp + 1]).reshape(_NC, _NS, per_t, _CH)
    dst3d = dstp.reshape(_NS, per_t, _CH)

    mesh = plsc.VectorSubcoreMesh(core_axis_name="c", subcore_axis_name="s",
                                  num_cores=_NC, num_subcores=_NS)
    sc_params = pltpu.CompilerParams(use_tc_tiling_on_sc=False)

    hist = pl.kernel(
        _deg_body,
        out_type=jax.ShapeDtypeStruct((_NC, NP), jnp.float32),
        mesh=mesh,
        scratch_types=[
            pltpu.VMEM((per_t // _NC, _CH), jnp.int32),
            pltpu.VMEM((_CH,), jnp.float32),
            pltpu.VMEM((NP // _NS,), jnp.float32),
            pltpu.VMEM_SHARED((NP,), jnp.float32),
        ] + [pltpu.SemaphoreType.DMA] * _NB,
        compiler_params=sc_params,
    )(dst3d)

    da = hist[0].reshape(NP, 1)
    db = hist[1].reshape(NP, 1)

    BL = 1000
    grid = (N // BL,)
    y, dis = pl.pallas_call(
        _tc1_body,
        grid=grid,
        in_specs=[
            pl.BlockSpec((BL, D), lambda i: (i, 0)),
            pl.BlockSpec((D, D), lambda i: (0, 0)),
            pl.BlockSpec((BL, 1), lambda i: (i, 0)),
            pl.BlockSpec((BL, 1), lambda i: (i, 0)),
        ],
        out_specs=[
            pl.BlockSpec((BL, D), lambda i: (i, 0)),
            pl.BlockSpec((BL, 1), lambda i: (i, 0)),
        ],
        out_shape=[
            jax.ShapeDtypeStruct((NP, D), jnp.float32),
            jax.ShapeDtypeStruct((NP, 1), jnp.float32),
        ],
    )(x, W, da, db)

    y2 = y.reshape(_NC * NP, HD)

    acc = pl.kernel(
        _edge_body,
        out_type=jax.ShapeDtypeStruct((_NC, NP, HD), jnp.float32),
        mesh=mesh,
        scratch_types=[
            pltpu.VMEM((per_t, _CH), jnp.int32),
            pltpu.VMEM((per_t, _CH), jnp.int32),
            pltpu.VMEM((_CH, HD), jnp.float32),
            pltpu.VMEM((_CH, HD), jnp.float32),
            pltpu.VMEM((_CH, HD), jnp.float32),
            pltpu.VMEM((_CH, HD), jnp.float32),
            pltpu.VMEM_SHARED((NP, HD), jnp.float32),
        ] + [pltpu.SemaphoreType.DMA] * (_NB + 1),
        compiler_params=sc_params,
    )(src4d, dst3d, y2)

    out = pl.pallas_call(
        _tc2_body,
        grid=grid,
        in_specs=[
            pl.BlockSpec((_NC, BL, HD), lambda i: (0, i, 0)),
            pl.BlockSpec((BL, 1), lambda i: (i, 0)),
            pl.BlockSpec((1, D), lambda i: (0, 0)),
        ],
        out_specs=pl.BlockSpec((BL, D), lambda i: (i, 0)),
        out_shape=jax.ShapeDtypeStruct((N, D), jnp.float32),
    )(acc, dis, b.reshape(1, D))

    return out
